# baseline (speedup 1.0000x reference)
import jax
import jax.numpy as jnp
from jax import lax
from jax.experimental import pallas as pl
from jax.experimental.pallas import tpu as pltpu
from jax.experimental.pallas import tpu_sc as plsc

_B, _S = 1024, 20
_N = _B * _S
_D = 4096
_NC, _NS = 2, 16
_NW = _NC * _NS
_BPW = _N // _NW
_C = 8


def kernel(indices, weight):
    idx_flat = indices.reshape(_N).astype(jnp.int32)

    mesh = plsc.VectorSubcoreMesh(
        core_axis_name="core", subcore_axis_name="subcore"
    )

    @pl.kernel(
        out_type=jax.ShapeDtypeStruct((_B, _S, _D), jnp.float32),
        mesh=mesh,
        compiler_params=pltpu.CompilerParams(use_tc_tiling_on_sc=True),
        scratch_types=[
            pltpu.VMEM((_BPW,), jnp.int32),
            pltpu.VMEM((_C, _D), jnp.float32),
            pltpu.VMEM((_C, _D), jnp.float32),
            pltpu.SemaphoreType.DMA,
            pltpu.SemaphoreType.DMA,
        ],
    )
    def gather_kernel(w_hbm, i_hbm, o_hbm, idx_v, buf0, buf1, sem0, sem1):
        wid = lax.axis_index("subcore") * _NC + lax.axis_index("core")
        base = wid * _BPW
        pltpu.sync_copy(i_hbm.at[pl.ds(base, _BPW)], idx_v)

        def start_gather(c, buf, sem):
            pltpu.make_async_copy(
                w_hbm.at[idx_v.at[pl.ds(c, _C)]], buf, sem
            ).start()

        def wait_gather(buf, sem):
            pltpu.make_async_copy(w_hbm.at[idx_v.at[pl.ds(0, _C)]], buf, sem).wait()

        def write_rows(c, buf):
            for k in range(_C):
                r = base + c + k
                pltpu.sync_copy(buf.at[k], o_hbm.at[r // _S, r % _S])

        start_gather(0, buf0, sem0)

        @pl.loop(0, _BPW, step=2 * _C)
        def _(c):
            wait_gather(buf0, sem0)
            start_gather(c + _C, buf1, sem1)
            write_rows(c, buf0)
            wait_gather(buf1, sem1)

            @pl.when(c + 2 * _C < _BPW)
            def _():
                start_gather(c + 2 * _C, buf0, sem0)

            write_rows(c + _C, buf1)

    return gather_kernel(weight, idx_flat)


# speedup vs baseline: 2.0601x; 2.0601x over previous
import jax
import jax.numpy as jnp
from jax import lax
from jax.experimental import pallas as pl
from jax.experimental.pallas import tpu as pltpu
from jax.experimental.pallas import tpu_sc as plsc

_B, _S = 1024, 20
_N = _B * _S
_D = 4096
_NC, _NS = 2, 16
_NW = _NC * _NS
_BPW = _N // _NW
_C = 8


def kernel(indices, weight):
    idx_flat = indices.reshape(_N).astype(jnp.int32)

    mesh = plsc.VectorSubcoreMesh(
        core_axis_name="core", subcore_axis_name="subcore"
    )

    @pl.kernel(
        out_type=jax.ShapeDtypeStruct((_S, _B, _D), jnp.float32),
        mesh=mesh,
        scratch_types=[
            pltpu.VMEM((_BPW,), jnp.int32),
            pltpu.VMEM((_C, _D), jnp.float32),
            pltpu.VMEM((_C, _D), jnp.float32),
            pltpu.SemaphoreType.DMA,
            pltpu.SemaphoreType.DMA,
        ],
    )
    def gather_kernel(w_hbm, i_hbm, o_hbm, idx_v, buf0, buf1, sem0, sem1):
        wid = lax.axis_index("subcore") * _NC + lax.axis_index("core")
        base = wid * _BPW
        pltpu.sync_copy(i_hbm.at[pl.ds(base, _BPW)], idx_v)

        def start_gather(c, buf, sem):
            pltpu.make_async_copy(
                w_hbm.at[idx_v.at[pl.ds(c, _C)]], buf, sem
            ).start()

        def wait_gather(buf, sem):
            pltpu.make_async_copy(w_hbm.at[idx_v.at[pl.ds(0, _C)]], buf, sem).wait()

        def write_rows(c, buf):
            for k in range(_C):
                r = base + c + k
                pltpu.sync_copy(buf.at[k], o_hbm.at[r % _S, r // _S])

        start_gather(0, buf0, sem0)

        @pl.loop(0, _BPW, step=2 * _C)
        def _(c):
            wait_gather(buf0, sem0)
            start_gather(c + _C, buf1, sem1)
            write_rows(c, buf0)
            wait_gather(buf1, sem1)

            @pl.when(c + 2 * _C < _BPW)
            def _():
                start_gather(c + 2 * _C, buf0, sem0)

            write_rows(c + _C, buf1)

    out_sbd = gather_kernel(weight, idx_flat)
    return jnp.transpose(out_sbd, (1, 0, 2))
